# baseline (speedup 1.0000x reference)
import functools

import jax
import jax.numpy as jnp
import numpy as np
from jax import lax
from jax.experimental import pallas as pl
from jax.experimental.pallas import tpu as pltpu
from jax.experimental.pallas import tpu_sc as plsc

_BATCH = 2048
_IN_DIM = 8192
_OUT_DIM = 8192
_TAU = 1.0

_NC = 2
_NS = 16
_NW = _NC * _NS
_L = 16

_ROWS_PER_W = _BATCH // _NW
_R = 3
_NCHUNK = _OUT_DIM // _L
_STEPS = [(3 * k, 3) for k in range(21)] + [(63, 1)]

_GATE_M = np.array(
    [
        [0.0, 0.0, 0.0, 0.0],
        [0.0, 0.0, 0.0, 1.0],
        [0.0, 1.0, 0.0, -1.0],
        [0.0, 1.0, 0.0, 0.0],
        [0.0, 0.0, 1.0, -1.0],
        [0.0, 0.0, 1.0, 0.0],
        [0.0, 1.0, 1.0, -2.0],
        [0.0, 1.0, 1.0, -1.0],
        [1.0, -1.0, -1.0, 1.0],
        [1.0, -1.0, -1.0, 2.0],
        [1.0, 0.0, -1.0, 0.0],
        [1.0, 0.0, -1.0, 1.0],
        [1.0, -1.0, 0.0, 0.0],
        [1.0, -1.0, 0.0, 1.0],
        [1.0, 0.0, 0.0, -1.0],
        [1.0, 0.0, 0.0, 0.0],
    ],
    dtype=np.float32,
)


def _coeff_body(w_ref, m_ref, out_ref):
    w = w_ref[...] * (1.0 / _TAU)
    m = jnp.max(w, axis=-1, keepdims=True)
    e = jnp.exp(w - m)
    s = e / jnp.sum(e, axis=-1, keepdims=True)
    out_ref[...] = jnp.dot(s, m_ref[...], preferred_element_type=jnp.float32,
                           precision=jax.lax.Precision.HIGHEST)


def _coeffs_tc(weight):
    out = pl.pallas_call(
        _coeff_body,
        out_shape=jax.ShapeDtypeStruct((_OUT_DIM, 4), jnp.float32),
    )(weight, jnp.asarray(_GATE_M))
    return out.T


def _sc_body(x_hbm, pc1_hbm, pc2_hbm, pk_hbm, out_hbm,
             pk_v, pc1_v, pc2_v,
             xin00, xin01, xin02, xin10, xin11, xin12, yout_v,
             in_sem0, in_sem1, out_sem0, out_sem1):
    xin_bufs = ((xin00, xin01, xin02), (xin10, xin11, xin12))
    wid = lax.axis_index("s") * _NC + lax.axis_index("c")
    row0 = wid * _ROWS_PER_W

    pltpu.sync_copy(pk_hbm, pk_v)
    pltpu.sync_copy(pc1_hbm, pc1_v)
    pltpu.sync_copy(pc2_hbm, pc2_v)

    in_sems = (in_sem0, in_sem1)
    out_sems = (out_sem0, out_sem1)

    def start_in(off_nrows, slot):
        off, nrows = off_nrows
        return [
            pltpu.async_copy(x_hbm.at[row0 + off + r], xin_bufs[slot][r],
                             in_sems[slot])
            for r in range(nrows)
        ]

    def start_out(off_nrows, slot):
        off, nrows = off_nrows
        return pltpu.async_copy(
            yout_v.at[slot, pl.ds(0, nrows)],
            out_hbm.at[pl.ds(row0 + off, nrows)],
            out_sems[slot])

    def compute(slot, nrows):
        @plsc.parallel_loop(0, _NCHUNK, 1, unroll=4)
        def _chunk(jc):
            off = jc * _L
            pab = pk_v[pl.ds(off, _L)]
            ia = jnp.bitwise_and(pab, 0xFFFF)
            ib = lax.shift_right_logical(pab, 16)
            p1 = pc1_v[pl.ds(off, _L)]
            p2 = pc2_v[pl.ds(off, _L)]
            c0 = plsc.bitcast(jnp.left_shift(p1, 16), jnp.float32)
            ca = plsc.bitcast(
                jnp.bitwise_and(p1, jnp.int32(-65536)), jnp.float32)
            cb = plsc.bitcast(jnp.left_shift(p2, 16), jnp.float32)
            cab = plsc.bitcast(
                jnp.bitwise_and(p2, jnp.int32(-65536)), jnp.float32)
            for r in range(nrows):
                a = plsc.load_gather(xin_bufs[slot][r], [ia])
                b = plsc.load_gather(xin_bufs[slot][r], [ib])
                yout_v[slot, r, pl.ds(off, _L)] = (
                    c0 + ca * a + cb * b + cab * (a * b))

    nsteps = len(_STEPS)
    d_in = {0: start_in(_STEPS[0], 0), 1: start_in(_STEPS[1], 1)}
    d_out = {}
    for g in range(nsteps):
        slot = g % 2
        if g >= 2:
            d_out[slot].wait()
        for d in d_in[slot]:
            d.wait()
        compute(slot, _STEPS[g][1])
        d_out[slot] = start_out(_STEPS[g], slot)
        if g + 2 < nsteps:
            d_in[slot] = start_in(_STEPS[g + 2], slot)
    d_out[(nsteps - 2) % 2].wait()
    d_out[(nsteps - 1) % 2].wait()


@functools.partial(jax.jit, donate_argnums=())
def _logic_sc(x, pc1, pc2, pk):
    mesh = plsc.VectorSubcoreMesh(
        core_axis_name="c", subcore_axis_name="s",
        num_cores=_NC, num_subcores=_NS)
    fn = pl.kernel(
        _sc_body,
        out_type=jax.ShapeDtypeStruct((_BATCH, _OUT_DIM), jnp.float32),
        mesh=mesh,
        compiler_params=pltpu.CompilerParams(needs_layout_passes=False, use_tc_tiling_on_sc=False),
        scratch_types=[
            pltpu.VMEM((_OUT_DIM,), jnp.int32),
            pltpu.VMEM((_OUT_DIM,), jnp.int32),
            pltpu.VMEM((_OUT_DIM,), jnp.int32),
            pltpu.VMEM((_IN_DIM,), jnp.float32),
            pltpu.VMEM((_IN_DIM,), jnp.float32),
            pltpu.VMEM((_IN_DIM,), jnp.float32),
            pltpu.VMEM((_IN_DIM,), jnp.float32),
            pltpu.VMEM((_IN_DIM,), jnp.float32),
            pltpu.VMEM((_IN_DIM,), jnp.float32),
            pltpu.VMEM((2, _R, _OUT_DIM), jnp.float32),
            pltpu.SemaphoreType.DMA,
            pltpu.SemaphoreType.DMA,
            pltpu.SemaphoreType.DMA,
            pltpu.SemaphoreType.DMA,
        ],
    )
    return fn(x, pc1, pc2, pk)


def _bf16_hi(v):
    b = lax.bitcast_convert_type(v.astype(jnp.bfloat16), jnp.uint16)
    return b.astype(jnp.int32)


def kernel(x, weight, idx_a, idx_b):
    coeffs = _coeffs_tc(weight)
    pc1 = jnp.bitwise_or(_bf16_hi(coeffs[0]),
                         jnp.left_shift(_bf16_hi(coeffs[1]), 16))
    pc2 = jnp.bitwise_or(_bf16_hi(coeffs[2]),
                         jnp.left_shift(_bf16_hi(coeffs[3]), 16))
    pk = jnp.bitwise_or(idx_a.astype(jnp.int32),
                        jnp.left_shift(idx_b.astype(jnp.int32), 16))
    return _logic_sc(x, pc1, pc2, pk)


# speedup vs baseline: 2.1081x; 2.1081x over previous
import functools

import jax
import jax.numpy as jnp
import numpy as np
from jax import lax
from jax.experimental import pallas as pl
from jax.experimental.pallas import tpu as pltpu
from jax.experimental.pallas import tpu_sc as plsc

_BATCH = 2048
_IN_DIM = 8192
_OUT_DIM = 8192
_TAU = 1.0

_NC = 2
_NS = 16
_NW = _NC * _NS
_L = 16

_ROWS_PER_W = _BATCH // _NW
_R = 3
_NCHUNK = _OUT_DIM // _L
_STEPS = [(3 * k, 3) for k in range(21)] + [(63, 1)]

_GATE_M = np.array(
    [
        [0.0, 0.0, 0.0, 0.0],
        [0.0, 0.0, 0.0, 1.0],
        [0.0, 1.0, 0.0, -1.0],
        [0.0, 1.0, 0.0, 0.0],
        [0.0, 0.0, 1.0, -1.0],
        [0.0, 0.0, 1.0, 0.0],
        [0.0, 1.0, 1.0, -2.0],
        [0.0, 1.0, 1.0, -1.0],
        [1.0, -1.0, -1.0, 1.0],
        [1.0, -1.0, -1.0, 2.0],
        [1.0, 0.0, -1.0, 0.0],
        [1.0, 0.0, -1.0, 1.0],
        [1.0, -1.0, 0.0, 0.0],
        [1.0, -1.0, 0.0, 1.0],
        [1.0, 0.0, 0.0, -1.0],
        [1.0, 0.0, 0.0, 0.0],
    ],
    dtype=np.float32,
)


def _coeff_body(w_ref, m_ref, out_ref):
    w = w_ref[...] * (1.0 / _TAU)
    m = jnp.max(w, axis=-1, keepdims=True)
    e = jnp.exp(w - m)
    s = e / jnp.sum(e, axis=-1, keepdims=True)
    out_ref[...] = jnp.dot(s, m_ref[...], preferred_element_type=jnp.float32,
                           precision=jax.lax.Precision.HIGHEST)


def _coeffs_tc(weight):
    out = pl.pallas_call(
        _coeff_body,
        out_shape=jax.ShapeDtypeStruct((_OUT_DIM, 4), jnp.float32),
    )(weight, jnp.asarray(_GATE_M))
    return out.T


def _sc_body(x_hbm, pc1_hbm, pc2_hbm, pk_hbm, out_hbm,
             pk_v, pc1_v, pc2_v,
             xin00, xin01, xin02, xin10, xin11, xin12,
             yo00, yo01, yo02, yo10, yo11, yo12,
             in_sem0, in_sem1, out_sem0, out_sem1):
    xin_bufs = ((xin00, xin01, xin02), (xin10, xin11, xin12))
    yout_bufs = ((yo00, yo01, yo02), (yo10, yo11, yo12))
    wid = lax.axis_index("s") * _NC + lax.axis_index("c")
    row0 = wid * _ROWS_PER_W

    pltpu.sync_copy(pk_hbm, pk_v)
    pltpu.sync_copy(pc1_hbm, pc1_v)
    pltpu.sync_copy(pc2_hbm, pc2_v)

    in_sems = (in_sem0, in_sem1)
    out_sems = (out_sem0, out_sem1)

    def start_in(off_nrows, slot):
        off, nrows = off_nrows
        return [
            pltpu.async_copy(x_hbm.at[row0 + off + r], xin_bufs[slot][r],
                             in_sems[slot])
            for r in range(nrows)
        ]

    def start_out(off_nrows, slot):
        off, nrows = off_nrows
        return [
            pltpu.async_copy(yout_bufs[slot][r], out_hbm.at[row0 + off + r],
                             out_sems[slot])
            for r in range(nrows)
        ]

    def compute(slot, nrows):
        @plsc.parallel_loop(0, _NCHUNK, 1, unroll=4)
        def _chunk(jc):
            off = jc * _L
            pab = pk_v[pl.ds(off, _L)]
            ia = jnp.bitwise_and(pab, 0xFFFF)
            ib = lax.shift_right_logical(pab, 16)
            p1 = pc1_v[pl.ds(off, _L)]
            p2 = pc2_v[pl.ds(off, _L)]
            c0 = plsc.bitcast(jnp.left_shift(p1, 16), jnp.float32)
            ca = plsc.bitcast(
                jnp.bitwise_and(p1, jnp.int32(-65536)), jnp.float32)
            cb = plsc.bitcast(jnp.left_shift(p2, 16), jnp.float32)
            cab = plsc.bitcast(
                jnp.bitwise_and(p2, jnp.int32(-65536)), jnp.float32)
            for r in range(nrows):
                a = plsc.load_gather(xin_bufs[slot][r], [ia])
                b = plsc.load_gather(xin_bufs[slot][r], [ib])
                yout_bufs[slot][r][pl.ds(off, _L)] = (
                    c0 + ca * a + cb * b + cab * (a * b))

    nsteps = len(_STEPS)
    d_in = {0: start_in(_STEPS[0], 0), 1: start_in(_STEPS[1], 1)}
    d_out = {}
    for g in range(nsteps):
        slot = g % 2
        if g >= 2:
            for d in d_out[slot]:
                d.wait()
        for d in d_in[slot]:
            d.wait()
        compute(slot, _STEPS[g][1])
        d_out[slot] = start_out(_STEPS[g], slot)
        if g + 2 < nsteps:
            d_in[slot] = start_in(_STEPS[g + 2], slot)
    for s in (0, 1):
        for d in d_out[s]:
            d.wait()


@functools.partial(jax.jit, donate_argnums=())
def _logic_sc(x, pc1, pc2, pk):
    mesh = plsc.VectorSubcoreMesh(
        core_axis_name="c", subcore_axis_name="s",
        num_cores=_NC, num_subcores=_NS)
    fn = pl.kernel(
        _sc_body,
        out_type=jax.ShapeDtypeStruct((_BATCH, _OUT_DIM), jnp.float32),
        mesh=mesh,
        compiler_params=pltpu.CompilerParams(needs_layout_passes=False),
        scratch_types=[
            pltpu.VMEM((_OUT_DIM,), jnp.int32),
            pltpu.VMEM((_OUT_DIM,), jnp.int32),
            pltpu.VMEM((_OUT_DIM,), jnp.int32),
            pltpu.VMEM((_IN_DIM,), jnp.float32),
            pltpu.VMEM((_IN_DIM,), jnp.float32),
            pltpu.VMEM((_IN_DIM,), jnp.float32),
            pltpu.VMEM((_IN_DIM,), jnp.float32),
            pltpu.VMEM((_IN_DIM,), jnp.float32),
            pltpu.VMEM((_IN_DIM,), jnp.float32),
            pltpu.VMEM((_OUT_DIM,), jnp.float32),
            pltpu.VMEM((_OUT_DIM,), jnp.float32),
            pltpu.VMEM((_OUT_DIM,), jnp.float32),
            pltpu.VMEM((_OUT_DIM,), jnp.float32),
            pltpu.VMEM((_OUT_DIM,), jnp.float32),
            pltpu.VMEM((_OUT_DIM,), jnp.float32),
            pltpu.SemaphoreType.DMA,
            pltpu.SemaphoreType.DMA,
            pltpu.SemaphoreType.DMA,
            pltpu.SemaphoreType.DMA,
        ],
    )
    return fn(x, pc1, pc2, pk)


def _bf16_hi(v):
    b = lax.bitcast_convert_type(v.astype(jnp.bfloat16), jnp.uint16)
    return b.astype(jnp.int32)


def kernel(x, weight, idx_a, idx_b):
    coeffs = _coeffs_tc(weight)
    pc1 = jnp.bitwise_or(_bf16_hi(coeffs[0]),
                         jnp.left_shift(_bf16_hi(coeffs[1]), 16))
    pc2 = jnp.bitwise_or(_bf16_hi(coeffs[2]),
                         jnp.left_shift(_bf16_hi(coeffs[3]), 16))
    pk = jnp.bitwise_or(idx_a.astype(jnp.int32),
                        jnp.left_shift(idx_b.astype(jnp.int32), 16))
    return _logic_sc(x, pc1, pc2, pk)
